# baseline (speedup 1.0000x reference)
import functools
import math

import jax
import jax.numpy as jnp
from jax import lax
from jax.experimental import pallas as pl
from jax.experimental.pallas import tpu as pltpu
from jax.experimental.pallas import tpu_sc as plsc

_NC = 2
_NS = 16
_NW = _NC * _NS

_D = 768
_GROUPS = _D // 16
_K = 32
_RING = 3


def _body(ids_hbm, table_hbm, pe_hbm, out_hbm,
          idx_v, pe_v, rows_v, sem_pe, sems_g, sems_s):
    batch, seq_len = ids_hbm.shape
    pos_per_w = seq_len // _NW
    n_chunks = pos_per_w // _K
    n_tasks = n_chunks * batch
    scale = jnp.float32(math.sqrt(float(_D)))

    w = lax.axis_index("s") * _NC + lax.axis_index("c")
    w_base = w * pos_per_w

    tasks = [(pc, b) for pc in range(n_chunks) for b in range(batch)]

    def issue_gather(t):
        pc, b = tasks[t]
        p = t % _RING
        return pltpu.async_copy(
            table_hbm.at[idx_v.at[b, pl.ds(pc * _K, _K)]],
            rows_v.at[p], sems_g[p])

    def issue_store(t):
        pc, b = tasks[t]
        p = t % _RING
        return pltpu.async_copy(
            rows_v.at[p],
            out_hbm.at[b, pl.ds(w_base + pc * _K, _K), :], sems_s[p])

    def compute(t):
        pc, _ = tasks[t]
        p = t % _RING
        pbuf = pc % 2

        def row_fma(i, carry):
            for j in range(_GROUPS):
                sl = pl.ds(j * 16, 16)
                rows_v[p, i, sl] = (rows_v[p, i, sl] * scale
                                    + pe_v[pbuf, i, sl])
            return carry
        lax.fori_loop(0, _K, row_fma, 0)

    for b in range(batch):
        pltpu.sync_copy(ids_hbm.at[b, pl.ds(w_base, pos_per_w)], idx_v.at[b])

    pe_cp0 = pltpu.async_copy(
        pe_hbm.at[pl.ds(w_base, _K), :], pe_v.at[0], sem_pe)
    gathers = {0: issue_gather(0), 1: issue_gather(1)}
    stores = {}
    pe_cp1 = None

    pe_cp0.wait()
    for t in range(n_tasks):
        gathers[t].wait()
        if t == 1:
            pe_cp1 = pltpu.async_copy(
                pe_hbm.at[pl.ds(w_base + _K, _K), :], pe_v.at[1], sem_pe)
        if t == batch:
            pe_cp1.wait()
        compute(t)
        if t + 2 < n_tasks:
            if t >= 1:
                stores[t - 1].wait()
            gathers[t + 2] = issue_gather(t + 2)
        stores[t] = issue_store(t)
    for t in range(n_tasks - _RING, n_tasks):
        stores[t].wait()


def kernel(input_ids, word_embeddings, pe):
    batch, seq_len = input_ids.shape
    ids32 = input_ids.astype(jnp.int32)
    pe2d = pe.reshape(pe.shape[1], pe.shape[2])
    pos_per_w = seq_len // _NW

    mesh = plsc.VectorSubcoreMesh(
        core_axis_name="c", subcore_axis_name="s",
        num_cores=_NC, num_subcores=_NS,
    )
    run = pl.kernel(
        _body,
        out_type=jax.ShapeDtypeStruct((batch, seq_len, _D), jnp.float32),
        mesh=mesh,
        scratch_types=[
            pltpu.VMEM((batch, pos_per_w), jnp.int32),
            pltpu.VMEM((2, _K, _D), jnp.float32),
            pltpu.VMEM((_RING, _K, _D), jnp.float32),
            pltpu.SemaphoreType.DMA,
            [pltpu.SemaphoreType.DMA] * _RING,
            [pltpu.SemaphoreType.DMA] * _RING,
        ],
    )
    return run(ids32, word_embeddings, pe2d)


# speedup vs baseline: 1.2708x; 1.2708x over previous
import functools
import math

import jax
import jax.numpy as jnp
from jax import lax
from jax.experimental import pallas as pl
from jax.experimental.pallas import tpu as pltpu
from jax.experimental.pallas import tpu_sc as plsc

_NC = 2
_NS = 16
_NW = _NC * _NS

_D = 768
_GROUPS = _D // 16


def _body(ids_hbm, table_hbm, pe_hbm, out_hbm, idx_v, pe_v, rows_v, sem):
    batch, seq_len = ids_hbm.shape
    pos_per_w = seq_len // _NW
    scale = jnp.float32(math.sqrt(float(_D)))

    w = lax.axis_index("s") * _NC + lax.axis_index("c")
    w_base = w * pos_per_w

    def row_fma(i, carry):
        for j in range(_GROUPS):
            sl = pl.ds(j * 16, 16)
            rows_v[i, sl] = rows_v[i, sl] * scale + pe_v[i, sl]
        return carry

    pltpu.sync_copy(pe_hbm.at[0, pl.ds(w_base, pos_per_w), :], pe_v)
    for b in range(batch):
        pltpu.sync_copy(ids_hbm.at[b, pl.ds(w_base, pos_per_w)], idx_v)
        pltpu.async_copy(table_hbm.at[idx_v], rows_v, sem).wait()
        lax.fori_loop(0, pos_per_w, row_fma, 0)
        pltpu.sync_copy(rows_v, out_hbm.at[b, pl.ds(w_base, pos_per_w), :])


def kernel(input_ids, word_embeddings, pe):
    batch, seq_len = input_ids.shape
    ids32 = input_ids.astype(jnp.int32)
    pos_per_w = seq_len // _NW

    mesh = plsc.VectorSubcoreMesh(
        core_axis_name="c", subcore_axis_name="s",
        num_cores=_NC, num_subcores=_NS,
    )
    run = pl.kernel(
        _body,
        out_type=jax.ShapeDtypeStruct((batch, seq_len, _D), jnp.float32),
        mesh=mesh,
        scratch_types=[
            pltpu.VMEM((pos_per_w,), jnp.int32),
            pltpu.VMEM((pos_per_w, _D), jnp.float32),
            pltpu.VMEM((pos_per_w, _D), jnp.float32),
            pltpu.SemaphoreType.DMA,
        ],
    )
    return run(ids32, word_embeddings, pe)
